# Initial kernel scaffold; baseline (speedup 1.0000x reference)
import jax
import jax.numpy as jnp
from jax.experimental import pallas as pl


def kernel(distance, scale):
    raise NotImplementedError("write your pallas kernel here")


# speedup vs baseline: 157.7915x; 157.7915x over previous
import functools

import numpy as np
import jax
import jax.numpy as jnp
from jax.experimental import pallas as pl
from jax.experimental.pallas import tpu as pltpu

H = 32
W = 32
M = 4096
IMG = 512
NUM_NN = 3
SIGMA = 4.0
KS = 2 * int(4.0 * SIGMA + 0.5) + 1


def _build_combined_matrix() -> np.ndarray:
    scale = IMG / H
    sample_f = (np.arange(IMG) + 0.5) / scale - 0.5
    x = np.abs(sample_f[None, :] - np.arange(H)[:, None])
    w = np.maximum(0.0, 1.0 - x)
    w = w / w.sum(axis=0, keepdims=True)
    R = w.T.astype(np.float64)

    xs = np.arange(KS, dtype=np.float64) - KS // 2
    k1 = np.exp(-(xs ** 2) / (2.0 * SIGMA ** 2))
    k1 = k1 / k1.sum()
    pad = KS // 2
    idx = np.arange(-pad, IMG + pad)
    ridx = np.where(idx < 0, -idx, np.where(idx >= IMG, 2 * IMG - 2 - idx, idx))
    G = np.zeros((IMG, IMG))
    for o in range(IMG):
        for t in range(KS):
            G[o, ridx[o + t]] += k1[t]
    return (G @ R).astype(np.float32)


_C_MATRIX = _build_combined_matrix()


def _score_block(x_ref, out_ref):
    x = x_ref[...]
    inf = jnp.float32(np.inf)

    m1 = jnp.min(x, axis=1, keepdims=True)
    eq1 = x == m1
    c1 = jnp.sum(eq1.astype(jnp.float32), axis=1, keepdims=True)
    y = jnp.where(eq1, inf, x)
    my = jnp.min(y, axis=1, keepdims=True)
    eqy = y == my
    cy = jnp.sum(eqy.astype(jnp.float32), axis=1, keepdims=True)
    z = jnp.where(eqy, inf, y)
    mz = jnp.min(z, axis=1, keepdims=True)

    m2 = jnp.where(c1 >= 2.0, m1, my)
    m3 = jnp.where(c1 >= 3.0, m1,
                   jnp.where(c1 == 2.0, my,
                             jnp.where(cy >= 2.0, my, mz)))

    d1 = jnp.sqrt(m1)
    d2 = jnp.sqrt(m2)
    d3 = jnp.sqrt(m3)
    denom = 1.0 + jnp.exp(d1 - d2) + jnp.exp(d1 - d3)
    out_ref[...] = d1 / denom


def _tail_block(s_ref, c_ref, out_ref):
    s = s_ref[0]
    c = c_ref[...]
    t = jax.lax.dot_general(c, s, (((1,), (0,)), ((), ())),
                            preferred_element_type=jnp.float32)
    out_ref[0] = jax.lax.dot_general(t, c, (((1,), (1,)), ((), ())),
                                     preferred_element_type=jnp.float32)


@functools.partial(jax.jit, static_argnames=())
def kernel(distance, scale):
    b = distance.shape[0]
    n = b * H * W
    flat = distance.reshape(n, M)

    rows = 256
    score = pl.pallas_call(
        _score_block,
        grid=(n // rows,),
        in_specs=[pl.BlockSpec((rows, M), lambda i: (i, 0))],
        out_specs=pl.BlockSpec((rows, 1), lambda i: (i, 0)),
        out_shape=jax.ShapeDtypeStruct((n, 1), jnp.float32),
    )(flat)

    s = score.reshape(b, H, W)
    cmat = jnp.asarray(_C_MATRIX)
    amap = pl.pallas_call(
        _tail_block,
        grid=(b,),
        in_specs=[
            pl.BlockSpec((1, H, W), lambda i: (i, 0, 0)),
            pl.BlockSpec((IMG, H), lambda i: (0, 0)),
        ],
        out_specs=pl.BlockSpec((1, IMG, IMG), lambda i: (i, 0, 0)),
        out_shape=jax.ShapeDtypeStruct((b, IMG, IMG), jnp.float32),
    )(s, cmat)

    del scale
    return amap.reshape(b, 1, IMG, IMG)


# speedup vs baseline: 199.9131x; 1.2669x over previous
import functools

import numpy as np
import jax
import jax.numpy as jnp
from jax.experimental import pallas as pl
from jax.experimental.pallas import tpu as pltpu

H = 32
W = 32
M = 4096
IMG = 512
NUM_NN = 3
SIGMA = 4.0
KS = 2 * int(4.0 * SIGMA + 0.5) + 1


def _build_combined_matrix() -> np.ndarray:
    scale = IMG / H
    sample_f = (np.arange(IMG) + 0.5) / scale - 0.5
    x = np.abs(sample_f[None, :] - np.arange(H)[:, None])
    w = np.maximum(0.0, 1.0 - x)
    w = w / w.sum(axis=0, keepdims=True)
    R = w.T.astype(np.float64)

    xs = np.arange(KS, dtype=np.float64) - KS // 2
    k1 = np.exp(-(xs ** 2) / (2.0 * SIGMA ** 2))
    k1 = k1 / k1.sum()
    pad = KS // 2
    idx = np.arange(-pad, IMG + pad)
    ridx = np.where(idx < 0, -idx, np.where(idx >= IMG, 2 * IMG - 2 - idx, idx))
    G = np.zeros((IMG, IMG))
    for o in range(IMG):
        for t in range(KS):
            G[o, ridx[o + t]] += k1[t]
    return (G @ R).astype(np.float32)


_C_MATRIX = _build_combined_matrix()


def _score_block(x_ref, out_ref):
    inf = jnp.float32(np.inf)
    a1 = x_ref[:, 0:128]
    a2 = jnp.full_like(a1, inf)
    a3 = a2
    for j in range(1, M // 128):
        v = x_ref[:, j * 128:(j + 1) * 128]
        t = jnp.maximum(a1, v)
        a1 = jnp.minimum(a1, v)
        a3 = jnp.minimum(a3, jnp.maximum(a2, t))
        a2 = jnp.minimum(a2, t)

    x = jnp.concatenate([a1, a2, a3], axis=1)
    m1 = jnp.min(x, axis=1, keepdims=True)
    eq1 = x == m1
    c1 = jnp.sum(eq1.astype(jnp.float32), axis=1, keepdims=True)
    y = jnp.where(eq1, inf, x)
    my = jnp.min(y, axis=1, keepdims=True)
    eqy = y == my
    cy = jnp.sum(eqy.astype(jnp.float32), axis=1, keepdims=True)
    z = jnp.where(eqy, inf, y)
    mz = jnp.min(z, axis=1, keepdims=True)

    m2 = jnp.where(c1 >= 2.0, m1, my)
    m3 = jnp.where(c1 >= 3.0, m1,
                   jnp.where(c1 == 2.0, my,
                             jnp.where(cy >= 2.0, my, mz)))

    d1 = jnp.sqrt(m1)
    d2 = jnp.sqrt(m2)
    d3 = jnp.sqrt(m3)
    denom = 1.0 + jnp.exp(d1 - d2) + jnp.exp(d1 - d3)
    out_ref[...] = d1 / denom


def _tail_block(s_ref, c_ref, out_ref):
    s = s_ref[0]
    c = c_ref[...]
    t = jax.lax.dot_general(c, s, (((1,), (0,)), ((), ())),
                            preferred_element_type=jnp.float32)
    out_ref[0] = jax.lax.dot_general(t, c, (((1,), (1,)), ((), ())),
                                     preferred_element_type=jnp.float32)


@functools.partial(jax.jit, static_argnames=())
def kernel(distance, scale):
    b = distance.shape[0]
    n = b * H * W
    flat = distance.reshape(n, M)

    rows = 256
    score = pl.pallas_call(
        _score_block,
        grid=(n // rows,),
        in_specs=[pl.BlockSpec((rows, M), lambda i: (i, 0))],
        out_specs=pl.BlockSpec((rows, 1), lambda i: (i, 0)),
        out_shape=jax.ShapeDtypeStruct((n, 1), jnp.float32),
    )(flat)

    s = score.reshape(b, H, W)
    cmat = jnp.asarray(_C_MATRIX)
    amap = pl.pallas_call(
        _tail_block,
        grid=(b,),
        in_specs=[
            pl.BlockSpec((1, H, W), lambda i: (i, 0, 0)),
            pl.BlockSpec((IMG, H), lambda i: (0, 0)),
        ],
        out_specs=pl.BlockSpec((1, IMG, IMG), lambda i: (i, 0, 0)),
        out_shape=jax.ShapeDtypeStruct((b, IMG, IMG), jnp.float32),
    )(s, cmat)

    del scale
    return amap.reshape(b, 1, IMG, IMG)


# speedup vs baseline: 230.2855x; 1.1519x over previous
import functools

import numpy as np
import jax
import jax.numpy as jnp
from jax.experimental import pallas as pl
from jax.experimental.pallas import tpu as pltpu

H = 32
W = 32
M = 4096
IMG = 512
NUM_NN = 3
SIGMA = 4.0
KS = 2 * int(4.0 * SIGMA + 0.5) + 1


def _build_combined_matrix() -> np.ndarray:
    scale = IMG / H
    sample_f = (np.arange(IMG) + 0.5) / scale - 0.5
    x = np.abs(sample_f[None, :] - np.arange(H)[:, None])
    w = np.maximum(0.0, 1.0 - x)
    w = w / w.sum(axis=0, keepdims=True)
    R = w.T.astype(np.float64)

    xs = np.arange(KS, dtype=np.float64) - KS // 2
    k1 = np.exp(-(xs ** 2) / (2.0 * SIGMA ** 2))
    k1 = k1 / k1.sum()
    pad = KS // 2
    idx = np.arange(-pad, IMG + pad)
    ridx = np.where(idx < 0, -idx, np.where(idx >= IMG, 2 * IMG - 2 - idx, idx))
    G = np.zeros((IMG, IMG))
    for o in range(IMG):
        for t in range(KS):
            G[o, ridx[o + t]] += k1[t]
    return (G @ R).astype(np.float32)


_C_MATRIX = _build_combined_matrix()


def _score_block(x_ref, out_ref):
    inf = jnp.float32(np.inf)
    a1 = x_ref[:, 0:128]
    a2 = jnp.full_like(a1, inf)
    a3 = a2
    for j in range(1, M // 128):
        v = x_ref[:, j * 128:(j + 1) * 128]
        t = jnp.maximum(a1, v)
        a1 = jnp.minimum(a1, v)
        a3 = jnp.minimum(a3, jnp.maximum(a2, t))
        a2 = jnp.minimum(a2, t)

    x = jnp.concatenate([a1, a2, a3], axis=1)
    m1 = jnp.min(x, axis=1, keepdims=True)
    eq1 = x == m1
    c1 = jnp.sum(eq1.astype(jnp.float32), axis=1, keepdims=True)
    y = jnp.where(eq1, inf, x)
    my = jnp.min(y, axis=1, keepdims=True)
    eqy = y == my
    cy = jnp.sum(eqy.astype(jnp.float32), axis=1, keepdims=True)
    z = jnp.where(eqy, inf, y)
    mz = jnp.min(z, axis=1, keepdims=True)

    m2 = jnp.where(c1 >= 2.0, m1, my)
    m3 = jnp.where(c1 >= 3.0, m1,
                   jnp.where(c1 == 2.0, my,
                             jnp.where(cy >= 2.0, my, mz)))

    d1 = jnp.sqrt(m1)
    d2 = jnp.sqrt(m2)
    d3 = jnp.sqrt(m3)
    denom = 1.0 + jnp.exp(d1 - d2) + jnp.exp(d1 - d3)
    out_ref[...] = d1 / denom


def _tail_block(s_ref, c_ref, out_ref):
    s = s_ref[0]
    c = c_ref[...]
    t = jax.lax.dot_general(c, s, (((1,), (0,)), ((), ())),
                            preferred_element_type=jnp.float32)
    out_ref[0] = jax.lax.dot_general(t, c, (((1,), (1,)), ((), ())),
                                     preferred_element_type=jnp.float32)


@functools.partial(jax.jit, static_argnames=())
def kernel(distance, scale):
    b = distance.shape[0]
    n = b * H * W
    flat = distance.reshape(n, M)

    rows = 512
    score = pl.pallas_call(
        _score_block,
        grid=(n // rows,),
        in_specs=[pl.BlockSpec((rows, M), lambda i: (i, 0))],
        out_specs=pl.BlockSpec((rows, 1), lambda i: (i, 0)),
        out_shape=jax.ShapeDtypeStruct((n, 1), jnp.float32),
    )(flat)

    s = score.reshape(b, H, W)
    cmat = jnp.asarray(_C_MATRIX)
    amap = pl.pallas_call(
        _tail_block,
        grid=(b,),
        in_specs=[
            pl.BlockSpec((1, H, W), lambda i: (i, 0, 0)),
            pl.BlockSpec((IMG, H), lambda i: (0, 0)),
        ],
        out_specs=pl.BlockSpec((1, IMG, IMG), lambda i: (i, 0, 0)),
        out_shape=jax.ShapeDtypeStruct((b, IMG, IMG), jnp.float32),
    )(s, cmat)

    del scale
    return amap.reshape(b, 1, IMG, IMG)


# speedup vs baseline: 244.6193x; 1.0622x over previous
import functools

import numpy as np
import jax
import jax.numpy as jnp
from jax.experimental import pallas as pl
from jax.experimental.pallas import tpu as pltpu

H = 32
W = 32
M = 4096
IMG = 512
NUM_NN = 3
SIGMA = 4.0
KS = 2 * int(4.0 * SIGMA + 0.5) + 1


def _build_combined_matrix() -> np.ndarray:
    scale = IMG / H
    sample_f = (np.arange(IMG) + 0.5) / scale - 0.5
    x = np.abs(sample_f[None, :] - np.arange(H)[:, None])
    w = np.maximum(0.0, 1.0 - x)
    w = w / w.sum(axis=0, keepdims=True)
    R = w.T.astype(np.float64)

    xs = np.arange(KS, dtype=np.float64) - KS // 2
    k1 = np.exp(-(xs ** 2) / (2.0 * SIGMA ** 2))
    k1 = k1 / k1.sum()
    pad = KS // 2
    idx = np.arange(-pad, IMG + pad)
    ridx = np.where(idx < 0, -idx, np.where(idx >= IMG, 2 * IMG - 2 - idx, idx))
    G = np.zeros((IMG, IMG))
    for o in range(IMG):
        for t in range(KS):
            G[o, ridx[o + t]] += k1[t]
    return (G @ R).astype(np.float32)


_C_MATRIX = _build_combined_matrix()


def _score_block(x_ref, out_ref):
    inf = jnp.float32(np.inf)
    a1 = x_ref[:, 0:128]
    a2 = jnp.full_like(a1, inf)
    a3 = a2
    for j in range(1, M // 128):
        v = x_ref[:, j * 128:(j + 1) * 128]
        t = jnp.maximum(a1, v)
        a1 = jnp.minimum(a1, v)
        a3 = jnp.minimum(a3, jnp.maximum(a2, t))
        a2 = jnp.minimum(a2, t)

    x = jnp.concatenate([a1, a2, a3], axis=1)
    m1 = jnp.min(x, axis=1, keepdims=True)
    eq1 = x == m1
    c1 = jnp.sum(eq1.astype(jnp.float32), axis=1, keepdims=True)
    y = jnp.where(eq1, inf, x)
    my = jnp.min(y, axis=1, keepdims=True)
    eqy = y == my
    cy = jnp.sum(eqy.astype(jnp.float32), axis=1, keepdims=True)
    z = jnp.where(eqy, inf, y)
    mz = jnp.min(z, axis=1, keepdims=True)

    m2 = jnp.where(c1 >= 2.0, m1, my)
    m3 = jnp.where(c1 >= 3.0, m1,
                   jnp.where(c1 == 2.0, my,
                             jnp.where(cy >= 2.0, my, mz)))

    d1 = jnp.sqrt(m1)
    d2 = jnp.sqrt(m2)
    d3 = jnp.sqrt(m3)
    denom = 1.0 + jnp.exp(d1 - d2) + jnp.exp(d1 - d3)
    out_ref[...] = d1 / denom


def _tail_block(s_ref, c_ref, out_ref):
    s = s_ref[0]
    c = c_ref[...]
    t = jax.lax.dot_general(c, s, (((1,), (0,)), ((), ())),
                            preferred_element_type=jnp.float32)
    out_ref[0] = jax.lax.dot_general(t, c, (((1,), (1,)), ((), ())),
                                     preferred_element_type=jnp.float32)


@functools.partial(jax.jit, static_argnames=())
def kernel(distance, scale):
    b = distance.shape[0]
    n = b * H * W
    flat = distance.reshape(n, M)

    rows = 1024
    score = pl.pallas_call(
        _score_block,
        grid=(n // rows,),
        in_specs=[pl.BlockSpec((rows, M), lambda i: (i, 0))],
        out_specs=pl.BlockSpec((rows, 1), lambda i: (i, 0)),
        out_shape=jax.ShapeDtypeStruct((n, 1), jnp.float32),
    )(flat)

    s = score.reshape(b, H, W)
    cmat = jnp.asarray(_C_MATRIX)
    amap = pl.pallas_call(
        _tail_block,
        grid=(b,),
        in_specs=[
            pl.BlockSpec((1, H, W), lambda i: (i, 0, 0)),
            pl.BlockSpec((IMG, H), lambda i: (0, 0)),
        ],
        out_specs=pl.BlockSpec((1, IMG, IMG), lambda i: (i, 0, 0)),
        out_shape=jax.ShapeDtypeStruct((b, IMG, IMG), jnp.float32),
    )(s, cmat)

    del scale
    return amap.reshape(b, 1, IMG, IMG)
